# Initial kernel scaffold; baseline (speedup 1.0000x reference)
import jax
import jax.numpy as jnp
from jax.experimental import pallas as pl


def kernel(weight):
    raise NotImplementedError("write your pallas kernel here")


# speedup vs baseline: 1.2744x; 1.2744x over previous
import jax
import jax.numpy as jnp
from jax.experimental import pallas as pl


_N_ROWS = 100000
_BLOCK_ROWS = 2000


def _copy_block(x_ref, o_ref):
    o_ref[...] = x_ref[...]


def kernel(weight):
    n, d = weight.shape
    grid = n // _BLOCK_ROWS
    return pl.pallas_call(
        _copy_block,
        grid=(grid,),
        in_specs=[pl.BlockSpec((_BLOCK_ROWS, d), lambda i: (i, 0))],
        out_specs=pl.BlockSpec((_BLOCK_ROWS, d), lambda i: (i, 0)),
        out_shape=jax.ShapeDtypeStruct((n, d), weight.dtype),
    )(weight)
